# baseline (speedup 1.0000x reference)
import functools

import jax
import jax.numpy as jnp
import numpy as np
from jax import lax
from jax.experimental import pallas as pl
from jax.experimental.pallas import tpu as pltpu
from jax.experimental.pallas import tpu_sc as plsc


def _scores_body(tok_ref, w1_ref, b1_ref, w2_ref, b2_ref, gok_ref,
                 out_ref, pad_ref, *, bb):
    w1 = w1_ref[...]
    b1 = b1_ref[...]
    w2 = w2_ref[...]
    b2 = b2_ref[0, 0]
    gok = gok_ref[0, 0] != 0
    w1b = w1.astype(jnp.bfloat16)
    w2b = w2.astype(jnp.bfloat16)
    for b in range(bb):
        xt = tok_ref[b]
        ht = jnp.dot(w1b, xt.astype(jnp.bfloat16),
                     preferred_element_type=jnp.float32) + b1
        g = 0.5 * ht * (1.0 + lax.erf(ht * np.float32(0.7071067811865476)))
        s = jnp.dot(w2b, g.astype(jnp.bfloat16),
                    preferred_element_type=jnp.float32) + b2
        out_ref[pl.ds(b, 1), :] = jnp.where(gok, s, jnp.nan)
        c, n = xt.shape
        pad_ref[b] = jnp.concatenate(
            [xt.T, jnp.zeros((n, 128 - c), jnp.float32)], axis=1)


def _scores_tc(tokens, fc1_w, fc1_b, fc2_w, fc2_b, gok, c0, nb):
    B, N, C = tokens.shape
    HID = fc1_w.shape[0]
    BB = 8
    b1 = fc1_b.reshape(HID, 1)
    w2 = fc2_w.reshape(1, HID)
    b2 = fc2_b.reshape(1, 1)
    return pl.pallas_call(
        functools.partial(_scores_body, bb=BB),
        grid=(nb,),
        in_specs=[
            pl.BlockSpec((BB, C, N), lambda i: (i + c0, 0, 0)),
            pl.BlockSpec((HID, C), lambda i: (0, 0)),
            pl.BlockSpec((HID, 1), lambda i: (0, 0)),
            pl.BlockSpec((1, HID), lambda i: (0, 0)),
            pl.BlockSpec((1, 1), lambda i: (0, 0)),
            pl.BlockSpec((1, 1), lambda i: (0, 0)),
        ],
        out_specs=[
            pl.BlockSpec((BB, N), lambda i: (i, 0)),
            pl.BlockSpec((BB, N, 128), lambda i: (i, 0, 0)),
        ],
        out_shape=[
            jax.ShapeDtypeStruct((nb * BB, N), jnp.float32),
            jax.ShapeDtypeStruct((nb * BB, N, 128), jnp.float32),
        ],
    )(jnp.transpose(tokens, (0, 2, 1)), fc1_w, b1, w2, b2, gok)


def _make_sc_kernel(B, Bc, N, C, K, row0):
    NW = 32
    RW = Bc // NW
    NV = N // 16
    KPAD = ((K + 15) // 16) * 16
    GPAD = KPAD
    GCH = 120
    NCH = GPAD // GCH
    mesh = plsc.VectorSubcoreMesh(core_axis_name="c", subcore_axis_name="s")

    @functools.partial(
        pl.kernel,
        mesh=mesh,
        compiler_params=pltpu.CompilerParams(needs_layout_passes=False),
        out_type=(),
        scratch_types=[
            pltpu.VMEM((1, N), jnp.float32),
            pltpu.VMEM((N,), jnp.uint32),
            pltpu.VMEM((256,), jnp.int32),
            pltpu.VMEM((GPAD,), jnp.int32),
            pltpu.VMEM((1, KPAD), jnp.int32),
            pltpu.VMEM((GPAD, 128), jnp.float32),
            pltpu.SemaphoreType.DMA,
        ],
    )
    def sc_kernel(scores_hbm, tokens_hbm, idx_hbm, pruned_hbm,
                  srow, ukeys, hist, gidx, lidx, rows, sem):
        wid = lax.axis_index("s") * 2 + lax.axis_index("c")
        iota = lax.iota(jnp.int32, 16)
        ones = jnp.ones((16,), jnp.int32)
        allt = iota >= 0

        def row_body(r, _):
            row = wid * RW + r
            pltpu.sync_copy(scores_hbm.at[row], srow)

            def mk(i, _):
                f = srow[0, pl.ds(i * 16, 16)]
                b = lax.bitcast_convert_type(f, jnp.uint32)
                neg = b >= jnp.uint32(0x80000000)
                ukeys[pl.ds(i * 16, 16)] = jnp.where(
                    neg, ~b, b ^ jnp.uint32(0x80000000))
                return 0
            lax.fori_loop(0, NV, mk, 0)

            P = jnp.uint32(0)
            kp = jnp.int32(K)
            for t in range(4):
                s = 24 - 8 * t

                def zero(j, _):
                    hist[pl.ds(j * 16, 16)] = jnp.zeros((16,), jnp.int32)
                    return 0
                lax.fori_loop(0, 16, zero, 0)

                def hacc(i, carry):
                    u = ukeys[pl.ds(i * 16, 16)]
                    digit = lax.shift_right_logical(
                        u, jnp.uint32(s)).astype(jnp.int32) & 255
                    if t == 0:
                        pm = allt
                    else:
                        pm = lax.shift_right_logical(
                            u, jnp.uint32(s + 8)) == carry
                    plsc.addupdate_scatter(hist, [digit], ones, mask=pm)
                    return carry
                lax.fori_loop(0, NV, hacc,
                              lax.shift_right_logical(P, jnp.uint32(s + 8)))

                def scan(jj, carry):
                    suffix, best = carry
                    j = 15 - jj
                    h = hist[pl.ds(j * 16, 16)]
                    S = jnp.sum(h)
                    c = plsc.cumsum(h)
                    ge = suffix + S - c + h
                    cand = jnp.max(jnp.where(ge >= kp, iota + 16 * j, -1))
                    return suffix + S, jnp.maximum(best, cand)
                _, d = lax.fori_loop(0, 16, scan,
                                     (jnp.int32(0), jnp.int32(-1)))

                def gtcnt(j, acc):
                    h = hist[pl.ds(j * 16, 16)]
                    return acc + jnp.sum(jnp.where(iota + 16 * j > d, h, 0))
                cnt_gt = lax.fori_loop(0, 16, gtcnt, jnp.int32(0))

                P = P | lax.shift_left(d.astype(jnp.uint32), jnp.uint32(s))
                kp = kp - cnt_gt
            T = P
            ntie = kp

            base = row * N

            def pad(j, _):
                gidx[pl.ds((KPAD - 16) + j * 16, 16)] = ones * base
                return 0
            lax.fori_loop(0, (GPAD - KPAD) // 16 + 1, pad, 0)

            def compact(i, carry):
                off, ties = carry
                u = ukeys[pl.ds(i * 16, 16)]
                gt = u > T
                eq = u == T
                eqi = eq.astype(jnp.int32)
                excl = plsc.cumsum(eqi) - eqi + ties
                keep = gt | (eq & (excl < ntie))
                ki = keep.astype(jnp.int32)
                pos = off + plsc.cumsum(ki) - ki
                plsc.store_scatter(gidx, [pos], iota + (16 * i + base),
                                   mask=keep)
                return off + jnp.sum(ki), ties + jnp.sum(eqi)
            lax.fori_loop(0, NV, compact, (jnp.int32(0), jnp.int32(0)))

            def loc(j, _):
                lidx[0, pl.ds(j * 16, 16)] = gidx[pl.ds(j * 16, 16)] - base
                return 0
            lax.fori_loop(0, KPAD // 16, loc, 0)
            pltpu.sync_copy(lidx, idx_hbm.at[row0 + row])

            copies = [
                pltpu.async_copy(
                    tokens_hbm.at[gidx.at[pl.ds(j * GCH, GCH)]],
                    rows.at[pl.ds(j * GCH, GCH)], sem)
                for j in range(NCH)
            ]
            for cp in copies:
                cp.wait()
            pltpu.sync_copy(rows.at[pl.ds(0, K)], pruned_hbm.at[row0 + row])
            return 0

        lax.fori_loop(0, RW, row_body, 0)

    return sc_kernel


def kernel(tokens, fc1_w, fc1_b, fc2_w, fc2_b, H, W):
    B, N, C = tokens.shape
    K = min(max(10, int(N * 0.7)), N)
    KPAD = ((K + 15) // 16) * 16

    gok = ((H * W) == N).astype(jnp.float32).reshape(1, 1)
    CH = 4
    Bc = B // CH
    idx_ref = jax.new_ref(lax.empty((B, 1, KPAD), jnp.int32))
    pruned_ref = jax.new_ref(lax.empty((B, K, 128), jnp.float32))
    scores_chunks = []
    for c in range(CH):
        scores_c, pad_c = _scores_tc(tokens, fc1_w, fc1_b, fc2_w, fc2_b,
                                     gok, c * (Bc // 8), Bc // 8)
        sc = _make_sc_kernel(B, Bc, N, C, K, c * Bc)
        sc(scores_c.reshape(Bc, 1, N), pad_c.reshape(Bc * N, 128),
           idx_ref, pruned_ref)
        scores_chunks.append(scores_c)
    scores = jnp.concatenate(scores_chunks, axis=0)
    idx_sorted = idx_ref[...].reshape(B, KPAD)[:, :K]
    return (pruned_ref[...][:, :, :C], idx_sorted, scores)


# speedup vs baseline: 1.0382x; 1.0382x over previous
import functools

import jax
import jax.numpy as jnp
import numpy as np
from jax import lax
from jax.experimental import pallas as pl
from jax.experimental.pallas import tpu as pltpu
from jax.experimental.pallas import tpu_sc as plsc


def _scores_body(tok_ref, w1_ref, b1_ref, w2_ref, b2_ref, gok_ref,
                 out_ref, pad_ref, *, bb):
    w1 = w1_ref[...]
    b1 = b1_ref[...]
    w2 = w2_ref[...]
    b2 = b2_ref[0, 0]
    gok = gok_ref[0, 0] != 0
    w1b = w1.astype(jnp.bfloat16)
    w2b = w2.astype(jnp.bfloat16)
    for b in range(bb):
        xt = tok_ref[b]
        ht = jnp.dot(w1b, xt.astype(jnp.bfloat16),
                     preferred_element_type=jnp.float32) + b1
        g = 0.5 * ht * (1.0 + lax.erf(ht * np.float32(0.7071067811865476)))
        s = jnp.dot(w2b, g.astype(jnp.bfloat16),
                    preferred_element_type=jnp.float32) + b2
        out_ref[pl.ds(b, 1), :] = jnp.where(gok, s, jnp.nan)
        c, n = xt.shape
        pad_ref[b] = jnp.concatenate(
            [xt.T, jnp.zeros((n, 128 - c), jnp.float32)], axis=1)


def _scores_tc(tokens, fc1_w, fc1_b, fc2_w, fc2_b, gok, c0, nb):
    B, N, C = tokens.shape
    HID = fc1_w.shape[0]
    BB = 8
    b1 = fc1_b.reshape(HID, 1)
    w2 = fc2_w.reshape(1, HID)
    b2 = fc2_b.reshape(1, 1)
    return pl.pallas_call(
        functools.partial(_scores_body, bb=BB),
        grid=(nb,),
        in_specs=[
            pl.BlockSpec((BB, C, N), lambda i: (i + c0, 0, 0)),
            pl.BlockSpec((HID, C), lambda i: (0, 0)),
            pl.BlockSpec((HID, 1), lambda i: (0, 0)),
            pl.BlockSpec((1, HID), lambda i: (0, 0)),
            pl.BlockSpec((1, 1), lambda i: (0, 0)),
            pl.BlockSpec((1, 1), lambda i: (0, 0)),
        ],
        out_specs=[
            pl.BlockSpec((BB, N), lambda i: (i, 0)),
            pl.BlockSpec((BB, N, 128), lambda i: (i, 0, 0)),
        ],
        out_shape=[
            jax.ShapeDtypeStruct((nb * BB, N), jnp.float32),
            jax.ShapeDtypeStruct((nb * BB, N, 128), jnp.float32),
        ],
    )(jnp.transpose(tokens, (0, 2, 1)), fc1_w, b1, w2, b2, gok)


def _make_sc_kernel(B, Bc, N, C, K, row0):
    NW = 32
    RW = Bc // NW
    NV = N // 16
    KPAD = ((K + 15) // 16) * 16
    GPAD = KPAD
    GCH = 120
    NCH = GPAD // GCH
    mesh = plsc.VectorSubcoreMesh(core_axis_name="c", subcore_axis_name="s")

    @functools.partial(
        pl.kernel,
        mesh=mesh,
        compiler_params=pltpu.CompilerParams(needs_layout_passes=False),
        out_type=(),
        scratch_types=[
            pltpu.VMEM((1, N), jnp.float32),
            pltpu.VMEM((N,), jnp.uint32),
            pltpu.VMEM((256,), jnp.int32),
            pltpu.VMEM((GPAD,), jnp.int32),
            pltpu.VMEM((GPAD,), jnp.int32),
            pltpu.VMEM((1, KPAD), jnp.int32),
            pltpu.VMEM((GPAD, 128), jnp.float32),
            pltpu.SemaphoreType.DMA,
        ],
    )
    def sc_kernel(scores_hbm, tokens_hbm, idx_hbm, pruned_hbm,
                  srow, ukeys, hist, gidx0, gidx1, lidx, rows, sem):
        wid = lax.axis_index("s") * 2 + lax.axis_index("c")
        iota = lax.iota(jnp.int32, 16)
        ones = jnp.ones((16,), jnp.int32)
        allt = iota >= 0

        def select_row(row, gidx):
            pltpu.sync_copy(scores_hbm.at[row], srow)

            def mk(i, _):
                f = srow[0, pl.ds(i * 16, 16)]
                b = lax.bitcast_convert_type(f, jnp.uint32)
                neg = b >= jnp.uint32(0x80000000)
                ukeys[pl.ds(i * 16, 16)] = jnp.where(
                    neg, ~b, b ^ jnp.uint32(0x80000000))
                return 0
            lax.fori_loop(0, NV, mk, 0)

            P = jnp.uint32(0)
            kp = jnp.int32(K)
            for t in range(4):
                s = 24 - 8 * t

                def zero(j, _):
                    hist[pl.ds(j * 16, 16)] = jnp.zeros((16,), jnp.int32)
                    return 0
                lax.fori_loop(0, 16, zero, 0)

                def hacc(i, carry):
                    u = ukeys[pl.ds(i * 16, 16)]
                    digit = lax.shift_right_logical(
                        u, jnp.uint32(s)).astype(jnp.int32) & 255
                    if t == 0:
                        pm = allt
                    else:
                        pm = lax.shift_right_logical(
                            u, jnp.uint32(s + 8)) == carry
                    plsc.addupdate_scatter(hist, [digit], ones, mask=pm)
                    return carry
                lax.fori_loop(0, NV, hacc,
                              lax.shift_right_logical(P, jnp.uint32(s + 8)))

                def scan(jj, carry):
                    suffix, best = carry
                    j = 15 - jj
                    h = hist[pl.ds(j * 16, 16)]
                    S = jnp.sum(h)
                    c = plsc.cumsum(h)
                    ge = suffix + S - c + h
                    cand = jnp.max(jnp.where(ge >= kp, iota + 16 * j, -1))
                    return suffix + S, jnp.maximum(best, cand)
                _, d = lax.fori_loop(0, 16, scan,
                                     (jnp.int32(0), jnp.int32(-1)))

                def gtcnt(j, acc):
                    h = hist[pl.ds(j * 16, 16)]
                    return acc + jnp.sum(jnp.where(iota + 16 * j > d, h, 0))
                cnt_gt = lax.fori_loop(0, 16, gtcnt, jnp.int32(0))

                P = P | lax.shift_left(d.astype(jnp.uint32), jnp.uint32(s))
                kp = kp - cnt_gt
            T = P
            ntie = kp

            base = row * N

            def pad(j, _):
                gidx[pl.ds((KPAD - 16) + j * 16, 16)] = ones * base
                return 0
            lax.fori_loop(0, (GPAD - KPAD) // 16 + 1, pad, 0)

            def compact(i, carry):
                off, ties = carry
                u = ukeys[pl.ds(i * 16, 16)]
                gt = u > T
                eq = u == T
                eqi = eq.astype(jnp.int32)
                excl = plsc.cumsum(eqi) - eqi + ties
                keep = gt | (eq & (excl < ntie))
                ki = keep.astype(jnp.int32)
                pos = off + plsc.cumsum(ki) - ki
                plsc.store_scatter(gidx, [pos], iota + (16 * i + base),
                                   mask=keep)
                return off + jnp.sum(ki), ties + jnp.sum(eqi)
            lax.fori_loop(0, NV, compact, (jnp.int32(0), jnp.int32(0)))

            def loc(j, _):
                lidx[0, pl.ds(j * 16, 16)] = gidx[pl.ds(j * 16, 16)] - base
                return 0
            lax.fori_loop(0, KPAD // 16, loc, 0)
            pltpu.sync_copy(lidx, idx_hbm.at[row0 + row])

        def fire_gathers(gidx):
            return [
                pltpu.async_copy(
                    tokens_hbm.at[gidx.at[pl.ds(j * GCH, GCH)]],
                    rows.at[pl.ds(j * GCH, GCH)], sem)
                for j in range(NCH)
            ]

        def drain_writeback(cps, row):
            half = NCH // 2
            h1 = half * GCH
            for cp in cps[:half]:
                cp.wait()
            pltpu.sync_copy(rows.at[pl.ds(0, h1)],
                            pruned_hbm.at[row0 + row, pl.ds(0, h1)])
            for cp in cps[half:]:
                cp.wait()
            pltpu.sync_copy(rows.at[pl.ds(h1, K - h1)],
                            pruned_hbm.at[row0 + row, pl.ds(h1, K - h1)])

        gbufs = (gidx0, gidx1)
        pending = None
        for r in range(RW):
            select_row(wid * RW + r, gbufs[r % 2])
            if pending is not None:
                drain_writeback(*pending)
            pending = (fire_gathers(gbufs[r % 2]), wid * RW + r)
        drain_writeback(*pending)

    return sc_kernel


def kernel(tokens, fc1_w, fc1_b, fc2_w, fc2_b, H, W):
    B, N, C = tokens.shape
    K = min(max(10, int(N * 0.7)), N)
    KPAD = ((K + 15) // 16) * 16

    gok = ((H * W) == N).astype(jnp.float32).reshape(1, 1)
    CH = 2
    Bc = B // CH
    idx_ref = jax.new_ref(lax.empty((B, 1, KPAD), jnp.int32))
    pruned_ref = jax.new_ref(lax.empty((B, K, 128), jnp.float32))
    scores_chunks = []
    for c in range(CH):
        scores_c, pad_c = _scores_tc(tokens, fc1_w, fc1_b, fc2_w, fc2_b,
                                     gok, c * (Bc // 8), Bc // 8)
        sc = _make_sc_kernel(B, Bc, N, C, K, c * Bc)
        sc(scores_c.reshape(Bc, 1, N), pad_c.reshape(Bc * N, 128),
           idx_ref, pruned_ref)
        scores_chunks.append(scores_c)
    scores = jnp.concatenate(scores_chunks, axis=0)
    idx_sorted = idx_ref[...].reshape(B, KPAD)[:, :K]
    return (pruned_ref[...][:, :, :C], idx_sorted, scores)


# speedup vs baseline: 1.0412x; 1.0029x over previous
import functools

import jax
import jax.numpy as jnp
import numpy as np
from jax import lax
from jax.experimental import pallas as pl
from jax.experimental.pallas import tpu as pltpu
from jax.experimental.pallas import tpu_sc as plsc


def _scores_body(tok_ref, w1_ref, b1_ref, w2_ref, b2_ref, gok_ref,
                 out_ref, pad_ref, *, bb):
    w1 = w1_ref[...]
    b1 = b1_ref[...]
    w2 = w2_ref[...]
    b2 = b2_ref[0, 0]
    gok = gok_ref[0, 0] != 0
    w1b = w1.astype(jnp.bfloat16)
    w2b = w2.astype(jnp.bfloat16)
    for b in range(bb):
        xt = tok_ref[b]
        ht = jnp.dot(w1b, xt.astype(jnp.bfloat16),
                     preferred_element_type=jnp.float32) + b1
        g = 0.5 * ht * (1.0 + lax.erf(ht * np.float32(0.7071067811865476)))
        s = jnp.dot(w2b, g.astype(jnp.bfloat16),
                    preferred_element_type=jnp.float32) + b2
        out_ref[b] = jnp.where(gok, s, jnp.nan)
        c, n = xt.shape
        pad_ref[b] = jnp.concatenate(
            [xt.T, jnp.zeros((n, 128 - c), jnp.float32)], axis=1)


def _scores_tc(tokens, fc1_w, fc1_b, fc2_w, fc2_b, gok, c0, nb):
    B, N, C = tokens.shape
    HID = fc1_w.shape[0]
    BB = 8
    b1 = fc1_b.reshape(HID, 1)
    w2 = fc2_w.reshape(1, HID)
    b2 = fc2_b.reshape(1, 1)
    return pl.pallas_call(
        functools.partial(_scores_body, bb=BB),
        grid=(nb,),
        in_specs=[
            pl.BlockSpec((BB, C, N), lambda i: (i + c0, 0, 0)),
            pl.BlockSpec((HID, C), lambda i: (0, 0)),
            pl.BlockSpec((HID, 1), lambda i: (0, 0)),
            pl.BlockSpec((1, HID), lambda i: (0, 0)),
            pl.BlockSpec((1, 1), lambda i: (0, 0)),
            pl.BlockSpec((1, 1), lambda i: (0, 0)),
        ],
        out_specs=[
            pl.BlockSpec((BB, 1, N), lambda i: (i, 0, 0)),
            pl.BlockSpec((BB, N, 128), lambda i: (i, 0, 0)),
        ],
        out_shape=[
            jax.ShapeDtypeStruct((nb * BB, 1, N), jnp.float32),
            jax.ShapeDtypeStruct((nb * BB, N, 128), jnp.float32),
        ],
    )(jnp.transpose(tokens, (0, 2, 1)), fc1_w, b1, w2, b2, gok)


def _make_sc_kernel(B, Bc, N, C, K, row0):
    NW = 32
    RW = Bc // NW
    NV = N // 16
    KPAD = ((K + 15) // 16) * 16
    GPAD = KPAD
    GCH = 120
    NCH = GPAD // GCH
    mesh = plsc.VectorSubcoreMesh(core_axis_name="c", subcore_axis_name="s")

    @functools.partial(
        pl.kernel,
        mesh=mesh,
        compiler_params=pltpu.CompilerParams(needs_layout_passes=False),
        out_type=(),
        scratch_types=[
            pltpu.VMEM((1, N), jnp.float32),
            pltpu.VMEM((N,), jnp.uint32),
            pltpu.VMEM((256,), jnp.int32),
            pltpu.VMEM((GPAD,), jnp.int32),
            pltpu.VMEM((GPAD,), jnp.int32),
            pltpu.VMEM((1, KPAD), jnp.int32),
            pltpu.VMEM((GPAD, 128), jnp.float32),
            pltpu.SemaphoreType.DMA,
        ],
    )
    def sc_kernel(scores_hbm, tokens_hbm, idx_hbm, pruned_hbm,
                  srow, ukeys, hist, gidx0, gidx1, lidx, rows, sem):
        wid = lax.axis_index("s") * 2 + lax.axis_index("c")
        iota = lax.iota(jnp.int32, 16)
        ones = jnp.ones((16,), jnp.int32)
        allt = iota >= 0

        def select_row(row, gidx):
            pltpu.sync_copy(scores_hbm.at[row], srow)

            def mk(i, _):
                f = srow[0, pl.ds(i * 16, 16)]
                b = lax.bitcast_convert_type(f, jnp.uint32)
                neg = b >= jnp.uint32(0x80000000)
                ukeys[pl.ds(i * 16, 16)] = jnp.where(
                    neg, ~b, b ^ jnp.uint32(0x80000000))
                return 0
            lax.fori_loop(0, NV, mk, 0)

            P = jnp.uint32(0)
            kp = jnp.int32(K)
            for t in range(4):
                s = 24 - 8 * t

                def zero(j, _):
                    hist[pl.ds(j * 16, 16)] = jnp.zeros((16,), jnp.int32)
                    return 0
                lax.fori_loop(0, 16, zero, 0)

                def hacc(i, carry):
                    u = ukeys[pl.ds(i * 16, 16)]
                    digit = lax.shift_right_logical(
                        u, jnp.uint32(s)).astype(jnp.int32) & 255
                    if t == 0:
                        pm = allt
                    else:
                        pm = lax.shift_right_logical(
                            u, jnp.uint32(s + 8)) == carry
                    plsc.addupdate_scatter(hist, [digit], ones, mask=pm)
                    return carry
                lax.fori_loop(0, NV, hacc,
                              lax.shift_right_logical(P, jnp.uint32(s + 8)))

                def scan(jj, carry):
                    suffix, best = carry
                    j = 15 - jj
                    h = hist[pl.ds(j * 16, 16)]
                    S = jnp.sum(h)
                    c = plsc.cumsum(h)
                    ge = suffix + S - c + h
                    cand = jnp.max(jnp.where(ge >= kp, iota + 16 * j, -1))
                    return suffix + S, jnp.maximum(best, cand)
                _, d = lax.fori_loop(0, 16, scan,
                                     (jnp.int32(0), jnp.int32(-1)))

                def gtcnt(j, acc):
                    h = hist[pl.ds(j * 16, 16)]
                    return acc + jnp.sum(jnp.where(iota + 16 * j > d, h, 0))
                cnt_gt = lax.fori_loop(0, 16, gtcnt, jnp.int32(0))

                P = P | lax.shift_left(d.astype(jnp.uint32), jnp.uint32(s))
                kp = kp - cnt_gt
            T = P
            ntie = kp

            base = row * N

            def pad(j, _):
                gidx[pl.ds((KPAD - 16) + j * 16, 16)] = ones * base
                return 0
            lax.fori_loop(0, (GPAD - KPAD) // 16 + 1, pad, 0)

            def compact(i, carry):
                off, ties = carry
                u = ukeys[pl.ds(i * 16, 16)]
                gt = u > T
                eq = u == T
                eqi = eq.astype(jnp.int32)
                excl = plsc.cumsum(eqi) - eqi + ties
                keep = gt | (eq & (excl < ntie))
                ki = keep.astype(jnp.int32)
                pos = off + plsc.cumsum(ki) - ki
                plsc.store_scatter(gidx, [pos], iota + (16 * i + base),
                                   mask=keep)
                return off + jnp.sum(ki), ties + jnp.sum(eqi)
            lax.fori_loop(0, NV, compact, (jnp.int32(0), jnp.int32(0)))

            def loc(j, _):
                lidx[0, pl.ds(j * 16, 16)] = gidx[pl.ds(j * 16, 16)] - base
                return 0
            lax.fori_loop(0, KPAD // 16, loc, 0)
            pltpu.sync_copy(lidx, idx_hbm.at[row0 + row])

        def fire_gathers(gidx):
            return [
                pltpu.async_copy(
                    tokens_hbm.at[gidx.at[pl.ds(j * GCH, GCH)]],
                    rows.at[pl.ds(j * GCH, GCH)], sem)
                for j in range(NCH)
            ]

        def drain_writeback(cps, row):
            half = NCH // 2
            h1 = half * GCH
            for cp in cps[:half]:
                cp.wait()
            pltpu.sync_copy(rows.at[pl.ds(0, h1)],
                            pruned_hbm.at[row0 + row, pl.ds(0, h1)])
            for cp in cps[half:]:
                cp.wait()
            pltpu.sync_copy(rows.at[pl.ds(h1, K - h1)],
                            pruned_hbm.at[row0 + row, pl.ds(h1, K - h1)])

        gbufs = (gidx0, gidx1)
        pending = None
        for r in range(RW):
            select_row(wid * RW + r, gbufs[r % 2])
            if pending is not None:
                drain_writeback(*pending)
            pending = (fire_gathers(gbufs[r % 2]), wid * RW + r)
        drain_writeback(*pending)

    return sc_kernel


def kernel(tokens, fc1_w, fc1_b, fc2_w, fc2_b, H, W):
    B, N, C = tokens.shape
    K = min(max(10, int(N * 0.7)), N)
    KPAD = ((K + 15) // 16) * 16

    gok = ((H * W) == N).astype(jnp.float32).reshape(1, 1)
    CH = 2
    Bc = B // CH
    idx_ref = jax.new_ref(lax.empty((B, 1, KPAD), jnp.int32))
    pruned_ref = jax.new_ref(lax.empty((B, K, 128), jnp.float32))
    scores_chunks = []
    for c in range(CH):
        scores_c, pad_c = _scores_tc(tokens, fc1_w, fc1_b, fc2_w, fc2_b,
                                     gok, c * (Bc // 8), Bc // 8)
        sc = _make_sc_kernel(B, Bc, N, C, K, c * Bc)
        sc(scores_c, pad_c.reshape(Bc * N, 128), idx_ref, pruned_ref)
        scores_chunks.append(scores_c)
    scores = jnp.concatenate(scores_chunks, axis=0).reshape(B, N)
    idx_sorted = idx_ref[...].reshape(B, KPAD)[:, :K]
    return (pruned_ref[...][:, :, :C], idx_sorted, scores)
